# baseline (speedup 1.0000x reference)
import functools

import jax
import jax.numpy as jnp
from jax import lax
from jax.experimental import pallas as pl
from jax.experimental.pallas import tpu as pltpu
from jax.experimental.pallas import tpu_sc as plsc

_B = 4096
_L = 50
_LP = 56
_EMB = 128
_NSPLIT = 4
_BS = _B // _NSPLIT

_info = plsc.get_sparse_core_info()
_NC = _info.num_cores
_NS = _info.num_subcores
_NW = _NC * _NS
_CHUNK = 128
_NCHUNK = _BS * _LP // (_NW * _CHUNK)
_PER_W = _NCHUNK * _CHUNK


@functools.partial(
    pl.kernel,
    mesh=plsc.VectorSubcoreMesh(core_axis_name="c", subcore_axis_name="s"),
    out_type=jax.ShapeDtypeStruct((_BS * _LP, _EMB), jnp.float32),
    scratch_types=[
        pltpu.VMEM((_NCHUNK, _CHUNK), jnp.int32),
        pltpu.VMEM((2, _CHUNK, _EMB), jnp.float32),
        pltpu.SemaphoreType.DMA,
        pltpu.SemaphoreType.DMA,
    ],
)
def _sc_gather(idx_hbm, table_hbm, out_hbm, idx_v, rows_v, sem0, sem1):
    wid = lax.axis_index("s") * _NC + lax.axis_index("c")
    base = wid * _PER_W
    pltpu.sync_copy(idx_hbm.at[wid], idx_v)

    sems = (sem0, sem1)

    def gather(j, b):
        return pltpu.make_async_copy(
            table_hbm.at[idx_v.at[j]], rows_v.at[b], sems[b])

    def writeback(j, b):
        pltpu.sync_copy(rows_v.at[b], out_hbm.at[pl.ds(base + j * _CHUNK, _CHUNK)])

    gather(0, 0).start()

    def body(g, carry):
        for b in range(2):
            j = 2 * g + b
            gather(j + 1, 1 - b).start()
            gather(j, b).wait()
            writeback(j, b)
        return carry

    lax.fori_loop(0, (_NCHUNK - 2) // 2, body, 0)

    gather(_NCHUNK - 1, 1).start()
    gather(_NCHUNK - 2, 0).wait()
    writeback(_NCHUNK - 2, 0)
    gather(_NCHUNK - 1, 1).wait()
    writeback(_NCHUNK - 1, 1)


def kernel(batch_word_indexes, batch_char_indexes, word_table):
    del batch_char_indexes
    vocab = word_table.shape[0]
    pad = (jax.lax.broadcasted_iota(jnp.int32, (_B, _LP - _L), 0) * (_LP - _L)
           + jax.lax.broadcasted_iota(jnp.int32, (_B, _LP - _L), 1)) % vocab
    idx = jnp.concatenate([batch_word_indexes, pad], axis=1)
    one = (batch_word_indexes[0, 0] * 0).astype(jnp.float32) + 1.0
    pieces = []
    for k in range(_NSPLIT):
        idx_k = lax.slice_in_dim(idx, k * _BS, (k + 1) * _BS, axis=0)
        out_k = _sc_gather(idx_k.reshape(_NW, _NCHUNK, _CHUNK), word_table)
        out_k = out_k.reshape(_BS, _LP, _EMB)[:, :_L, :] * one
        pieces.append(out_k)
    return jnp.concatenate(pieces, axis=0)


# speedup vs baseline: 1.5600x; 1.5600x over previous
import functools

import jax
import jax.numpy as jnp
from jax import lax
from jax.experimental import pallas as pl
from jax.experimental.pallas import tpu as pltpu
from jax.experimental.pallas import tpu_sc as plsc

_B = 4096
_L = 50
_LP = 56
_EMB = 128

_info = plsc.get_sparse_core_info()
_NC = _info.num_cores
_NS = _info.num_subcores
_NW = _NC * _NS
_ROWS_W = _B // _NW
_IDX_W = _ROWS_W * _LP


@functools.partial(
    pl.kernel,
    mesh=plsc.VectorSubcoreMesh(core_axis_name="c", subcore_axis_name="s"),
    out_type=jax.ShapeDtypeStruct((_B, _L, _EMB), jnp.float32),
    compiler_params=pltpu.CompilerParams(use_tc_tiling_on_sc=True),
    scratch_types=[
        pltpu.VMEM((_IDX_W,), jnp.int32),
        pltpu.VMEM((2, _LP, _EMB), jnp.float32),
        pltpu.SemaphoreType.DMA,
        pltpu.SemaphoreType.DMA,
    ],
)
def _sc_gather(idx_hbm, table_hbm, out_hbm, idx_v, rows_v, sem0, sem1):
    wid = lax.axis_index("s") * _NC + lax.axis_index("c")
    wbase = wid * _ROWS_W
    pltpu.sync_copy(idx_hbm.at[pl.ds(wid * _IDX_W, _IDX_W)], idx_v)

    sems = (sem0, sem1)

    def gather(r, b):
        return pltpu.make_async_copy(
            table_hbm.at[idx_v.at[pl.ds(r * _LP, _LP)]], rows_v.at[b], sems[b])

    def writeback(r, b):
        pltpu.sync_copy(rows_v.at[b, pl.ds(0, _L)], out_hbm.at[wbase + r])

    gather(0, 0).start()

    def body(g, carry):
        for b in range(2):
            r = 2 * g + b
            gather(r + 1, 1 - b).start()
            gather(r, b).wait()
            writeback(r, b)
        return carry

    lax.fori_loop(0, (_ROWS_W - 2) // 2, body, 0)

    gather(_ROWS_W - 1, 1).start()
    gather(_ROWS_W - 2, 0).wait()
    writeback(_ROWS_W - 2, 0)
    gather(_ROWS_W - 1, 1).wait()
    writeback(_ROWS_W - 1, 1)


def kernel(batch_word_indexes, batch_char_indexes, word_table):
    del batch_char_indexes
    vocab = word_table.shape[0]
    pad = (jax.lax.broadcasted_iota(jnp.int32, (_B, _LP - _L), 0) * (_LP - _L)
           + jax.lax.broadcasted_iota(jnp.int32, (_B, _LP - _L), 1)) % vocab
    idx = jnp.concatenate([batch_word_indexes, pad], axis=1).reshape(-1)
    return _sc_gather(idx, word_table)


# speedup vs baseline: 3.2164x; 2.0619x over previous
import functools

import jax
import jax.numpy as jnp
from jax import lax
from jax.experimental import pallas as pl
from jax.experimental.pallas import tpu as pltpu
from jax.experimental.pallas import tpu_sc as plsc

_B = 4096
_L = 50
_EMB = 128
_TOTAL = _B * _L

_info = plsc.get_sparse_core_info()
_NC = _info.num_cores
_NS = _info.num_subcores
_NW = _NC * _NS
_PER_W = _TOTAL // _NW
_CHUNK = 128
_NCHUNK = _PER_W // _CHUNK


@functools.partial(
    pl.kernel,
    mesh=plsc.VectorSubcoreMesh(core_axis_name="c", subcore_axis_name="s"),
    out_type=jax.ShapeDtypeStruct((_TOTAL, _EMB), jnp.float32),
    scratch_types=[
        pltpu.VMEM((_NCHUNK, _CHUNK), jnp.int32),
        pltpu.VMEM((2, _CHUNK, _EMB), jnp.float32),
        pltpu.SemaphoreType.DMA,
        pltpu.SemaphoreType.DMA,
    ],
)
def _sc_gather(idx_hbm, table_hbm, out_hbm, idx_v, rows_v, sem0, sem1):
    wid = lax.axis_index("s") * _NC + lax.axis_index("c")
    base = wid * _PER_W
    pltpu.sync_copy(idx_hbm.at[wid], idx_v)

    sems = (sem0, sem1)

    def gather(j, b):
        return pltpu.make_async_copy(
            table_hbm.at[idx_v.at[j]], rows_v.at[b], sems[b])

    def writeback(j, b):
        pltpu.sync_copy(rows_v.at[b], out_hbm.at[pl.ds(base + j * _CHUNK, _CHUNK)])

    gather(0, 0).start()

    def body(g, carry):
        for b in range(2):
            j = 2 * g + b
            gather(j + 1, 1 - b).start()
            gather(j, b).wait()
            writeback(j, b)
        return carry

    lax.fori_loop(0, (_NCHUNK - 2) // 2, body, 0)

    gather(_NCHUNK - 1, 1).start()
    gather(_NCHUNK - 2, 0).wait()
    writeback(_NCHUNK - 2, 0)
    gather(_NCHUNK - 1, 1).wait()
    writeback(_NCHUNK - 1, 1)


def kernel(batch_word_indexes, batch_char_indexes, word_table):
    del batch_char_indexes
    idx = batch_word_indexes.T.reshape(_NW, _NCHUNK, _CHUNK)
    out = _sc_gather(idx, word_table)
    return out.reshape(_L, _B, _EMB).transpose(1, 0, 2)


# speedup vs baseline: 3.2241x; 1.0024x over previous
import functools

import jax
import jax.numpy as jnp
from jax import lax
from jax.experimental import pallas as pl
from jax.experimental.pallas import tpu as pltpu
from jax.experimental.pallas import tpu_sc as plsc

_B = 4096
_L = 50
_EMB = 128
_TOTAL = _B * _L

_info = plsc.get_sparse_core_info()
_NC = _info.num_cores
_NS = _info.num_subcores
_NW = _NC * _NS
_PER_W = _TOTAL // _NW
_CHUNK = 256
_NCHUNK = _PER_W // _CHUNK


@functools.partial(
    pl.kernel,
    mesh=plsc.VectorSubcoreMesh(core_axis_name="c", subcore_axis_name="s"),
    out_type=jax.ShapeDtypeStruct((_TOTAL, _EMB), jnp.float32),
    scratch_types=[
        pltpu.VMEM((_PER_W,), jnp.int32),
        pltpu.VMEM((2, _CHUNK, _EMB), jnp.float32),
        pltpu.SemaphoreType.DMA,
        pltpu.SemaphoreType.DMA,
    ],
)
def _sc_gather(idx_hbm, table_hbm, out_hbm, idx_v, rows_v, sem0, sem1):
    wid = lax.axis_index("s") * _NC + lax.axis_index("c")
    base = wid * _PER_W
    pltpu.sync_copy(idx_hbm.at[pl.ds(base, _PER_W)], idx_v)

    sems = (sem0, sem1)

    def gather(j, b):
        return pltpu.make_async_copy(
            table_hbm.at[idx_v.at[pl.ds(j * _CHUNK, _CHUNK)]], rows_v.at[b],
            sems[b])

    def writeback(j, b):
        pltpu.sync_copy(rows_v.at[b], out_hbm.at[pl.ds(base + j * _CHUNK, _CHUNK)])

    gather(0, 0).start()

    def body(g, carry):
        for b in range(2):
            j = 2 * g + b
            gather(j + 1, 1 - b).start()
            gather(j, b).wait()
            writeback(j, b)
        return carry

    lax.fori_loop(0, (_NCHUNK - 1) // 2, body, 0)

    gather(_NCHUNK - 1, 0).wait()
    writeback(_NCHUNK - 1, 0)


def kernel(batch_word_indexes, batch_char_indexes, word_table):
    del batch_char_indexes
    idx = batch_word_indexes.T.reshape(-1)
    out = _sc_gather(idx, word_table)
    return out.reshape(_L, _B, _EMB).transpose(1, 0, 2)


# speedup vs baseline: 3.2281x; 1.0012x over previous
import functools

import jax
import jax.numpy as jnp
from jax import lax
from jax.experimental import pallas as pl
from jax.experimental.pallas import tpu as pltpu
from jax.experimental.pallas import tpu_sc as plsc

_B = 4096
_L = 50
_EMB = 128
_TOTAL = _B * _L

_info = plsc.get_sparse_core_info()
_NC = _info.num_cores
_NS = _info.num_subcores
_NW = _NC * _NS
_PER_W = _TOTAL // _NW
_CHUNK = 128
_NCHUNK = _PER_W // _CHUNK


@functools.partial(
    pl.kernel,
    mesh=plsc.VectorSubcoreMesh(core_axis_name="c", subcore_axis_name="s"),
    out_type=jax.ShapeDtypeStruct((_TOTAL, _EMB), jnp.float32),
    scratch_types=[
        pltpu.VMEM((_PER_W,), jnp.int32),
        pltpu.VMEM((4, _CHUNK, _EMB), jnp.float32),
        pltpu.SemaphoreType.DMA,
        pltpu.SemaphoreType.DMA,
        pltpu.SemaphoreType.DMA,
        pltpu.SemaphoreType.DMA,
        pltpu.SemaphoreType.DMA,
        pltpu.SemaphoreType.DMA,
        pltpu.SemaphoreType.DMA,
        pltpu.SemaphoreType.DMA,
    ],
)
def _sc_gather(idx_hbm, table_hbm, out_hbm, idx_v, rows_v,
               g0, g1, g2, g3, w0, w1, w2, w3):
    wid = lax.axis_index("s") * _NC + lax.axis_index("c")
    base = wid * _PER_W
    pltpu.sync_copy(idx_hbm.at[pl.ds(base, _PER_W)], idx_v)

    gs = (g0, g1, g2, g3)
    ws = (w0, w1, w2, w3)

    def gather(j, b):
        return pltpu.make_async_copy(
            table_hbm.at[idx_v.at[pl.ds(j * _CHUNK, _CHUNK)]], rows_v.at[b],
            gs[b])

    def writeback(j, b):
        return pltpu.make_async_copy(
            rows_v.at[b], out_hbm.at[pl.ds(base + j * _CHUNK, _CHUNK)], ws[b])

    gather(0, 0).start()
    gather(1, 1).start()
    gather(0, 0).wait()
    writeback(0, 0).start()
    gather(2, 2).start()
    gather(1, 1).wait()
    writeback(1, 1).start()
    gather(3, 3).start()

    def body(carry_g, carry):
        for bp in range(4):
            j = 4 * carry_g + 2 + bp
            b = (2 + bp) % 4
            gather(j, b).wait()
            writeback(j, b).start()
            writeback(j - 2, bp).wait()
            gather(j + 2, bp).start()
        return carry

    lax.fori_loop(0, (_NCHUNK - 6) // 4, body, 0)

    j = _NCHUNK - 4
    gather(j, 2).wait()
    writeback(j, 2).start()
    writeback(j - 2, 0).wait()
    gather(j + 2, 0).start()
    gather(j + 1, 3).wait()
    writeback(j + 1, 3).start()
    writeback(j - 1, 1).wait()
    gather(j + 3, 1).start()
    gather(j + 2, 0).wait()
    writeback(j + 2, 0).start()
    gather(j + 3, 1).wait()
    writeback(j + 3, 1).start()
    writeback(j, 2).wait()
    writeback(j + 1, 3).wait()
    writeback(j + 2, 0).wait()
    writeback(j + 3, 1).wait()


def kernel(batch_word_indexes, batch_char_indexes, word_table):
    del batch_char_indexes
    idx = batch_word_indexes.T.reshape(-1)
    out = _sc_gather(idx, word_table)
    return out.reshape(_L, _B, _EMB).transpose(1, 0, 2)


# speedup vs baseline: 3.2458x; 1.0055x over previous
import functools

import jax
import jax.numpy as jnp
from jax import lax
from jax.experimental import pallas as pl
from jax.experimental.pallas import tpu as pltpu
from jax.experimental.pallas import tpu_sc as plsc

_B = 4096
_L = 50
_EMB = 128
_TOTAL = _B * _L

_info = plsc.get_sparse_core_info()
_NC = _info.num_cores
_NS = _info.num_subcores
_NW = _NC * _NS
_CHUNK = _B // _NW


@functools.partial(
    pl.kernel,
    mesh=plsc.VectorSubcoreMesh(core_axis_name="c", subcore_axis_name="s"),
    out_type=jax.ShapeDtypeStruct((_TOTAL, _EMB), jnp.float32),
    scratch_types=[
        pltpu.VMEM((_L, _CHUNK), jnp.int32),
        pltpu.VMEM((2, _CHUNK, _EMB), jnp.float32),
        pltpu.SemaphoreType.DMA,
        pltpu.SemaphoreType.DMA,
    ],
)
def _sc_gather(idx_hbm, table_hbm, out_hbm, idx_v, rows_v, sem0, sem1):
    wid = lax.axis_index("s") * _NC + lax.axis_index("c")
    base = wid * _CHUNK
    pltpu.sync_copy(idx_hbm.at[:, pl.ds(base, _CHUNK)], idx_v)

    sems = (sem0, sem1)

    def gather(l, b):
        return pltpu.make_async_copy(
            table_hbm.at[idx_v.at[l]], rows_v.at[b], sems[b])

    def writeback(l, b):
        pltpu.sync_copy(rows_v.at[b], out_hbm.at[pl.ds(l * _B + base, _CHUNK)])

    gather(0, 0).start()

    def body(g, carry):
        for b in range(2):
            l = 2 * g + b
            gather(l + 1, 1 - b).start()
            gather(l, b).wait()
            writeback(l, b)
        return carry

    lax.fori_loop(0, (_L - 2) // 2, body, 0)

    gather(_L - 1, 1).start()
    gather(_L - 2, 0).wait()
    writeback(_L - 2, 0)
    gather(_L - 1, 1).wait()
    writeback(_L - 1, 1)


def kernel(batch_word_indexes, batch_char_indexes, word_table):
    del batch_char_indexes
    out = _sc_gather(batch_word_indexes.T, word_table)
    return out.reshape(_L, _B, _EMB).transpose(1, 0, 2)


# speedup vs baseline: 3.3274x; 1.0252x over previous
import functools

import jax
import jax.numpy as jnp
from jax import lax
from jax.experimental import pallas as pl
from jax.experimental.pallas import tpu as pltpu
from jax.experimental.pallas import tpu_sc as plsc

_B = 4096
_L = 50
_EMB = 128
_TOTAL = _B * _L

_info = plsc.get_sparse_core_info()
_NC = _info.num_cores
_NS = _info.num_subcores
_NW = _NC * _NS
_CHUNK = _B // _NW


@functools.partial(
    pl.kernel,
    mesh=plsc.VectorSubcoreMesh(core_axis_name="c", subcore_axis_name="s"),
    out_type=jax.ShapeDtypeStruct((_TOTAL, _EMB), jnp.float32),
    scratch_types=[
        pltpu.VMEM((_L, _CHUNK), jnp.int32),
        pltpu.VMEM((4, _CHUNK, _EMB), jnp.float32),
        pltpu.SemaphoreType.DMA,
        pltpu.SemaphoreType.DMA,
        pltpu.SemaphoreType.DMA,
        pltpu.SemaphoreType.DMA,
        pltpu.SemaphoreType.DMA,
        pltpu.SemaphoreType.DMA,
        pltpu.SemaphoreType.DMA,
        pltpu.SemaphoreType.DMA,
    ],
)
def _sc_gather(idx_hbm, table_hbm, out_hbm, idx_v, rows_v,
               g0, g1, g2, g3, w0, w1, w2, w3):
    wid = lax.axis_index("s") * _NC + lax.axis_index("c")
    base = wid * _CHUNK
    pltpu.sync_copy(idx_hbm.at[:, pl.ds(base, _CHUNK)], idx_v)

    gs = (g0, g1, g2, g3)
    ws = (w0, w1, w2, w3)

    def gather(l, b):
        return pltpu.make_async_copy(
            table_hbm.at[idx_v.at[l]], rows_v.at[b], gs[b])

    def writeback(l, b):
        return pltpu.make_async_copy(
            rows_v.at[b], out_hbm.at[pl.ds(l * _B + base, _CHUNK)], ws[b])

    gather(0, 0).start()
    gather(1, 1).start()
    gather(0, 0).wait()
    writeback(0, 0).start()
    gather(2, 2).start()
    gather(1, 1).wait()
    writeback(1, 1).start()
    gather(3, 3).start()

    def body(g, carry):
        for bp in range(4):
            l = 4 * g + 2 + bp
            b = (2 + bp) % 4
            gather(l, b).wait()
            writeback(l, b).start()
            writeback(l - 2, bp).wait()
            gather(l + 2, bp).start()
        return carry

    lax.fori_loop(0, (_L - 6) // 4, body, 0)

    l = _L - 4
    gather(l, 2).wait()
    writeback(l, 2).start()
    writeback(l - 2, 0).wait()
    gather(l + 2, 0).start()
    gather(l + 1, 3).wait()
    writeback(l + 1, 3).start()
    writeback(l - 1, 1).wait()
    gather(l + 3, 1).start()
    gather(l + 2, 0).wait()
    writeback(l + 2, 0).start()
    gather(l + 3, 1).wait()
    writeback(l + 3, 1).start()
    writeback(l, 2).wait()
    writeback(l + 1, 3).wait()
    writeback(l + 2, 0).wait()
    writeback(l + 3, 1).wait()


def kernel(batch_word_indexes, batch_char_indexes, word_table):
    del batch_char_indexes
    out = _sc_gather(batch_word_indexes.T, word_table)
    return out.reshape(_L, _B, _EMB).transpose(1, 0, 2)
